# baseline (speedup 1.0000x reference)
import functools

import jax
import jax.numpy as jnp
from jax import lax
from jax.experimental import pallas as pl
from jax.experimental.pallas import tpu as pltpu
from jax.experimental.pallas import tpu_sc as plsc

_NUM_P = 20000
_NUM_G = 500
_GPAD = 512
_LANES = 16

_ROWS_BIG = 632
_ROWS_SMALL = 624
_NBIG = 4
_CHUNK = 64
_TAIL_BIG = _ROWS_BIG - 9 * _CHUNK
_TAIL_SMALL = _ROWS_SMALL - 9 * _CHUNK


def _coef_body(gt_ref, out_ref):
    cx = gt_ref[0:1, :]
    cy = gt_ref[1:2, :]
    w = gt_ref[2:3, :]
    h = gt_ref[3:4, :]
    ang = gt_ref[4:5, :]
    cos = jnp.cos(ang)
    sin = jnp.sin(ang)
    ia = (2.0 / w) ** 2
    ib = (2.0 / h) ** 2
    a_c = cos * cos * ia + sin * sin * ib
    c_c = sin * sin * ia + cos * cos * ib
    b_c = 2.0 * cos * sin * (ia - ib)
    l1 = -(2.0 * a_c * cx + b_c * cy)
    l2 = -(2.0 * c_c * cy + b_c * cx)
    k_c = a_c * cx * cx + c_c * cy * cy + b_c * cx * cy
    out_ref[...] = jnp.zeros((8, _GPAD), jnp.float32)
    out_ref[0:1, 0:_NUM_G] = a_c
    out_ref[1:2, 0:_NUM_G] = c_c
    out_ref[2:3, 0:_NUM_G] = b_c
    out_ref[3:4, 0:_NUM_G] = l1
    out_ref[4:5, 0:_NUM_G] = l2
    out_ref[5:6, 0:_NUM_G] = k_c


def _coef_table(gt_bboxes):
    return pl.pallas_call(
        _coef_body,
        out_shape=jax.ShapeDtypeStruct((8, _GPAD), jnp.float32),
    )(gt_bboxes.T)


def _sc_body(g_hbm, pts_hbm, out_hbm,
             a_v, c_v, b_v, l1_v, l2_v, k_v, buf_v, pts_s):
    wid = lax.axis_index("s") * 2 + lax.axis_index("c")
    is_big = wid < _NBIG
    row0 = jnp.where(is_big,
                     wid * _ROWS_BIG,
                     _NBIG * _ROWS_BIG + (wid - _NBIG) * _ROWS_SMALL)
    row0 = pl.multiple_of(row0, 8)

    coef_refs = (a_v, c_v, b_v, l1_v, l2_v, k_v)
    for i, ref in enumerate(coef_refs):
        pltpu.sync_copy(g_hbm.at[i], ref)

    @pl.when(is_big)
    def _():
        src = pl.multiple_of(row0 * 2, 8)
        pltpu.sync_copy(pts_hbm.at[pl.ds(src, _ROWS_BIG * 2)],
                        pts_s.at[pl.ds(0, _ROWS_BIG * 2)])

    @pl.when(jnp.logical_not(is_big))
    def _():
        src = pl.multiple_of(row0 * 2, 8)
        pltpu.sync_copy(pts_hbm.at[pl.ds(src, _ROWS_SMALL * 2)],
                        pts_s.at[pl.ds(0, _ROWS_SMALL * 2)])

    chunk_offs = [16 * c for c in range(31)] + [484]

    def do_chunk(i, carry):
        def point_pass(offs):
            coefs = [[ref[pl.ds(o, _LANES)] for ref in coef_refs]
                     for o in offs]

            def body(q, pcarry):
                pv = pts_s[pl.ds((i * _CHUNK + q * 8) * 2, _LANES)]
                for k in range(8):
                    px = pv[2 * k]
                    py = pv[2 * k + 1]
                    px2 = px * px
                    py2 = py * py
                    pxpy = px * py
                    base = (q * 8 + k) * _NUM_G
                    for (av, cv, bv, l1v, l2v, kv), o in zip(coefs, offs):
                        acc = (av * px2 + cv * py2 + bv * pxpy
                               + l1v * px + l2v * py + kv)
                        buf_v[pl.ds(base + o, _LANES)] = acc
                return pcarry

            lax.fori_loop(0, _CHUNK // 8, body, 0)

        for g in range(8):
            point_pass(chunk_offs[4 * g:4 * g + 4])

        @pl.when(i < 9)
        def _():
            start = pl.multiple_of((row0 + i * _CHUNK) * _NUM_G, 8)
            pltpu.sync_copy(buf_v.at[pl.ds(0, _CHUNK * _NUM_G)],
                            out_hbm.at[pl.ds(start, _CHUNK * _NUM_G)])

        @pl.when(jnp.logical_and(i == 9, is_big))
        def _():
            start = pl.multiple_of((row0 + 9 * _CHUNK) * _NUM_G, 8)
            n = _TAIL_BIG * _NUM_G
            pltpu.sync_copy(buf_v.at[pl.ds(0, n)],
                            out_hbm.at[pl.ds(start, n)])

        @pl.when(jnp.logical_and(i == 9, jnp.logical_not(is_big)))
        def _():
            start = pl.multiple_of((row0 + 9 * _CHUNK) * _NUM_G, 8)
            n = _TAIL_SMALL * _NUM_G
            pltpu.sync_copy(buf_v.at[pl.ds(0, n)],
                            out_hbm.at[pl.ds(start, n)])

        return carry

    lax.fori_loop(0, 10, do_chunk, 0)


def _sc_map(g_tab, points):
    mesh = plsc.VectorSubcoreMesh(core_axis_name="c", subcore_axis_name="s")
    f = functools.partial(
        pl.kernel,
        mesh=mesh,
        out_type=jax.ShapeDtypeStruct((_NUM_P * _NUM_G,), jnp.float32),
        scratch_types=[
            pltpu.VMEM((_GPAD,), jnp.float32),
            pltpu.VMEM((_GPAD,), jnp.float32),
            pltpu.VMEM((_GPAD,), jnp.float32),
            pltpu.VMEM((_GPAD,), jnp.float32),
            pltpu.VMEM((_GPAD,), jnp.float32),
            pltpu.VMEM((_GPAD,), jnp.float32),
            pltpu.VMEM((_CHUNK * _NUM_G,), jnp.float32),
            pltpu.VMEM((_CHUNK * 10 * 2,), jnp.float32),
        ],
    )(_sc_body)
    return f(g_tab, points.reshape(-1))


def kernel(gt_bboxes, points):
    g_tab = _coef_table(gt_bboxes)
    flat = _sc_map(g_tab, points)
    return flat.reshape(_NUM_P, _NUM_G)


# speedup vs baseline: 2.1529x; 2.1529x over previous
import functools

import jax
import jax.numpy as jnp
from jax import lax
from jax.experimental import pallas as pl
from jax.experimental.pallas import tpu as pltpu
from jax.experimental.pallas import tpu_sc as plsc

_NUM_P = 20000
_NUM_G = 500
_GPAD = 512
_LANES = 16

_ROWS_BIG = 632
_ROWS_SMALL = 624
_NBIG = 4
_CHUNK = 64
_TAIL_BIG = _ROWS_BIG - 9 * _CHUNK
_TAIL_SMALL = _ROWS_SMALL - 9 * _CHUNK


def _coef_body(gt_ref, out_ref):
    cx = gt_ref[0:1, :]
    cy = gt_ref[1:2, :]
    w = gt_ref[2:3, :]
    h = gt_ref[3:4, :]
    ang = gt_ref[4:5, :]
    cos = jnp.cos(ang)
    sin = jnp.sin(ang)
    ia = (2.0 / w) ** 2
    ib = (2.0 / h) ** 2
    a_c = cos * cos * ia + sin * sin * ib
    c_c = sin * sin * ia + cos * cos * ib
    b_c = 2.0 * cos * sin * (ia - ib)
    l1 = -(2.0 * a_c * cx + b_c * cy)
    l2 = -(2.0 * c_c * cy + b_c * cx)
    k_c = a_c * cx * cx + c_c * cy * cy + b_c * cx * cy
    out_ref[...] = jnp.zeros((8, _GPAD), jnp.float32)
    out_ref[0:1, 0:_NUM_G] = a_c
    out_ref[1:2, 0:_NUM_G] = c_c
    out_ref[2:3, 0:_NUM_G] = b_c
    out_ref[3:4, 0:_NUM_G] = l1
    out_ref[4:5, 0:_NUM_G] = l2
    out_ref[5:6, 0:_NUM_G] = k_c


def _coef_table(gt_bboxes):
    return pl.pallas_call(
        _coef_body,
        out_shape=jax.ShapeDtypeStruct((8, _GPAD), jnp.float32),
    )(gt_bboxes.T)


def _sc_body(g_hbm, pts_hbm, out_hbm,
             a_v, c_v, b_v, l1_v, l2_v, k_v, buf_v, pts_s):
    wid = lax.axis_index("s") * 2 + lax.axis_index("c")
    is_big = wid < _NBIG
    row0 = jnp.where(is_big,
                     wid * _ROWS_BIG,
                     _NBIG * _ROWS_BIG + (wid - _NBIG) * _ROWS_SMALL)
    row0 = pl.multiple_of(row0, 8)

    coef_refs = (a_v, c_v, b_v, l1_v, l2_v, k_v)
    for i, ref in enumerate(coef_refs):
        pltpu.sync_copy(g_hbm.at[i], ref)

    @pl.when(is_big)
    def _():
        src = pl.multiple_of(row0 * 2, 8)
        pltpu.sync_copy(pts_hbm.at[pl.ds(src, _ROWS_BIG * 2)],
                        pts_s.at[pl.ds(0, _ROWS_BIG * 2)])

    @pl.when(jnp.logical_not(is_big))
    def _():
        src = pl.multiple_of(row0 * 2, 8)
        pltpu.sync_copy(pts_hbm.at[pl.ds(src, _ROWS_SMALL * 2)],
                        pts_s.at[pl.ds(0, _ROWS_SMALL * 2)])

    chunk_offs = [16 * c for c in range(31)] + [484]

    def do_chunk(i, carry):
        def point_pass(offs):
            coefs = [[ref[pl.ds(o, _LANES)] for ref in coef_refs]
                     for o in offs]

            def body(q, pcarry):
                pv = pts_s[pl.ds((i * _CHUNK + q * 8) * 2, _LANES)]
                for k in range(8):
                    px = pv[2 * k]
                    py = pv[2 * k + 1]
                    px2 = px * px
                    py2 = py * py
                    pxpy = px * py
                    row = q * 8 + k
                    for (av, cv, bv, l1v, l2v, kv), o in zip(coefs, offs):
                        acc = (av * px2 + cv * py2 + bv * pxpy
                               + l1v * px + l2v * py + kv)
                        buf_v[row, pl.ds(o, _LANES)] = acc
                return pcarry

            lax.fori_loop(0, _CHUNK // 8, body, 0)

        for g in range(8):
            point_pass(chunk_offs[4 * g:4 * g + 4])

        @pl.when(i < 9)
        def _():
            start = pl.multiple_of(row0 + i * _CHUNK, 8)
            pltpu.sync_copy(buf_v.at[pl.ds(0, _CHUNK)],
                            out_hbm.at[pl.ds(start, _CHUNK)])

        @pl.when(jnp.logical_and(i == 9, is_big))
        def _():
            start = pl.multiple_of(row0 + 9 * _CHUNK, 8)
            pltpu.sync_copy(buf_v.at[pl.ds(0, _TAIL_BIG)],
                            out_hbm.at[pl.ds(start, _TAIL_BIG)])

        @pl.when(jnp.logical_and(i == 9, jnp.logical_not(is_big)))
        def _():
            start = pl.multiple_of(row0 + 9 * _CHUNK, 8)
            pltpu.sync_copy(buf_v.at[pl.ds(0, _TAIL_SMALL)],
                            out_hbm.at[pl.ds(start, _TAIL_SMALL)])

        return carry

    lax.fori_loop(0, 10, do_chunk, 0)


def _sc_map(g_tab, points):
    mesh = plsc.VectorSubcoreMesh(core_axis_name="c", subcore_axis_name="s")
    f = functools.partial(
        pl.kernel,
        mesh=mesh,
        out_type=jax.ShapeDtypeStruct((_NUM_P, _NUM_G), jnp.float32),
        scratch_types=[
            pltpu.VMEM((_GPAD,), jnp.float32),
            pltpu.VMEM((_GPAD,), jnp.float32),
            pltpu.VMEM((_GPAD,), jnp.float32),
            pltpu.VMEM((_GPAD,), jnp.float32),
            pltpu.VMEM((_GPAD,), jnp.float32),
            pltpu.VMEM((_GPAD,), jnp.float32),
            pltpu.VMEM((_CHUNK, _NUM_G), jnp.float32),
            pltpu.VMEM((_CHUNK * 10 * 2,), jnp.float32),
        ],
    )(_sc_body)
    return f(g_tab, points.reshape(-1))


def kernel(gt_bboxes, points):
    g_tab = _coef_table(gt_bboxes)
    return _sc_map(g_tab, points)
